# baseline (speedup 1.0000x reference)
import functools

import jax
import jax.numpy as jnp
from jax.experimental import pallas as pl

TOKENS = 32768
DIM = 768
N_EXPERTS = 8
LANES = 128
BLOCK = 4096

NEG_INF = float("-inf")


def _gate_kernel(x_ref, wt_ref, out_ref):
    wt = wt_ref[...]
    s = jnp.dot(x_ref[...], wt, preferred_element_type=jnp.float32)
    lane = jax.lax.broadcasted_iota(jnp.int32, s.shape, 1)
    s = jnp.where(lane < N_EXPERTS, s, NEG_INF)
    m = jnp.max(s, axis=1, keepdims=True)
    denom = jnp.sum(jnp.exp(s - m), axis=1, keepdims=True)
    idx_f = jnp.argmax(s, axis=1).reshape(-1, 1).astype(jnp.float32)
    out_ref[...] = jnp.concatenate([1.0 / denom, idx_f], axis=1)


@jax.jit
def kernel(x, weight):
    wt = jnp.zeros((DIM, LANES), dtype=jnp.float32).at[:, :N_EXPERTS].set(
        weight.T)
    grid = (TOKENS // BLOCK,)
    packed = pl.pallas_call(
        _gate_kernel,
        grid=grid,
        in_specs=[
            pl.BlockSpec((BLOCK, DIM), lambda i: (i, 0)),
            pl.BlockSpec((DIM, LANES), lambda i: (0, 0)),
        ],
        out_specs=pl.BlockSpec((BLOCK, 2), lambda i: (i, 0)),
        out_shape=jax.ShapeDtypeStruct((TOKENS, 2), jnp.float32),
    )(x, wt)
    return packed[:, :1], packed[:, 1:2].astype(jnp.int32)


# speedup vs baseline: 1.3241x; 1.3241x over previous
import functools

import jax
import jax.numpy as jnp
from jax.experimental import pallas as pl

TOKENS = 32768
DIM = 768
N_EXPERTS = 8
LANES = 128
BLOCK = 4096

NEG_INF = float("-inf")


def _gate_kernel(x_ref, w_ref, w_out_ref, idx_out_ref):
    s = jax.lax.dot_general(
        x_ref[...], w_ref[...],
        dimension_numbers=(((1,), (1,)), ((), ())),
        preferred_element_type=jnp.float32)
    m = jnp.max(s, axis=1, keepdims=True)
    denom = jnp.sum(jnp.exp(s - m), axis=1, keepdims=True)
    w_out_ref[...] = 1.0 / denom
    idx_out_ref[...] = jnp.argmax(s, axis=1).reshape(-1, 1).astype(jnp.int32)


@jax.jit
def kernel(x, weight):
    grid = (TOKENS // BLOCK,)
    weights, indices = pl.pallas_call(
        _gate_kernel,
        grid=grid,
        in_specs=[
            pl.BlockSpec((BLOCK, DIM), lambda i: (i, 0)),
            pl.BlockSpec((N_EXPERTS, DIM), lambda i: (0, 0)),
        ],
        out_specs=[
            pl.BlockSpec((BLOCK, 1), lambda i: (i, 0)),
            pl.BlockSpec((BLOCK, 1), lambda i: (i, 0)),
        ],
        out_shape=[
            jax.ShapeDtypeStruct((TOKENS, 1), jnp.float32),
            jax.ShapeDtypeStruct((TOKENS, 1), jnp.int32),
        ],
    )(x, weight)
    return weights, indices


# speedup vs baseline: 2.2827x; 1.7240x over previous
import functools

import jax
import jax.numpy as jnp
from jax.experimental import pallas as pl

TOKENS = 32768
DIM = 768
N_EXPERTS = 8
LANES = 128
BLOCK = 8192

NEG_INF = float("-inf")


def _gate_kernel(x_ref, w_ref, w_out_ref, idx_out_ref):
    s = jax.lax.dot_general(
        x_ref[...], w_ref[...],
        dimension_numbers=(((1,), (1,)), ((), ())),
        preferred_element_type=jnp.float32)
    st = s.T
    m = jnp.max(st, axis=0, keepdims=True)
    denom = jnp.sum(jnp.exp(st - m), axis=0, keepdims=True)
    w_out_ref[...] = (1.0 / denom).reshape(BLOCK)
    idx_out_ref[...] = jnp.argmax(st, axis=0).reshape(BLOCK).astype(jnp.int32)


@jax.jit
def kernel(x, weight):
    grid = (TOKENS // BLOCK,)
    weights, indices = pl.pallas_call(
        _gate_kernel,
        grid=grid,
        in_specs=[
            pl.BlockSpec((BLOCK, DIM), lambda i: (i, 0)),
            pl.BlockSpec((N_EXPERTS, DIM), lambda i: (0, 0)),
        ],
        out_specs=[
            pl.BlockSpec((BLOCK,), lambda i: (i,)),
            pl.BlockSpec((BLOCK,), lambda i: (i,)),
        ],
        out_shape=[
            jax.ShapeDtypeStruct((TOKENS,), jnp.float32),
            jax.ShapeDtypeStruct((TOKENS,), jnp.int32),
        ],
    )(x, weight)
    return weights.reshape(TOKENS, 1), indices.reshape(TOKENS, 1)
